# baseline (speedup 1.0000x reference)
import functools

import jax
import jax.numpy as jnp
from jax import lax
from jax.experimental import pallas as pl
from jax.experimental.pallas import tpu as pltpu
from jax.experimental.pallas import tpu_sc as plsc

_NUM_CORES = 2
_NUM_SUBCORES = 16
_NUM_WORKERS = _NUM_CORES * _NUM_SUBCORES
_LANES = 16
_BC = 128
_TD = 4
_SG = 8
_PRE = 4


def _gather_call(S, V):
    row_words = _TD * 8 * _BC
    g_rows = _SG * _BC
    n_groups = S // _SG
    mesh = plsc.VectorSubcoreMesh(core_axis_name="c", subcore_axis_name="s")

    @functools.partial(
        pl.kernel,
        mesh=mesh,
        out_type=jax.ShapeDtypeStruct((S, _NUM_WORKERS * row_words),
                                      jnp.float32),
        scratch_types=(
            [pltpu.VMEM((S * _BC,), jnp.int32)]
            + [pltpu.VMEM((2 * g_rows, 32), jnp.float32)]
            + [pltpu.VMEM((2 * row_words,), jnp.float32)]
            + [pltpu.VMEM((_BC * 33,), jnp.float32)]
            + [pltpu.SemaphoreType.DMA for _ in range(2)]
        ),
        compiler_params=pltpu.CompilerParams(use_tc_tiling_on_sc=False,
                                             needs_layout_passes=False),
    )
    def gather_kernel(table_hbm, ids_hbm, out_hbm, idx_all, rows_v, out_v,
                      pad_v, sem_g, sem_o):
        w = lax.axis_index("s") * _NUM_CORES + lax.axis_index("c")

        pltpu.sync_copy(ids_hbm.at[w], idx_all)

        def start_gather(p):
            half = lax.rem(p, 2) * g_rows
            pltpu.async_copy(
                table_hbm.at[idx_all.at[pl.ds(p * g_rows, g_rows)]],
                rows_v.at[pl.ds(half, g_rows), :], sem_g)

        def wait_gather():
            pltpu.make_async_copy(
                table_hbm.at[idx_all.at[pl.ds(0, g_rows)]],
                rows_v.at[pl.ds(0, g_rows), :], sem_g).wait()

        def start_out(ob, s):
            for td in range(_TD):
                pltpu.async_copy(
                    out_v.at[pl.ds(ob * row_words + td * 1024, 1024)],
                    out_hbm.at[s, pl.ds(td * _NUM_WORKERS * 1024 + w * 1024,
                                        1024)],
                    sem_o)

        def wait_out():
            pltpu.make_async_copy(out_v.at[pl.ds(0, row_words)],
                                  out_hbm.at[0, pl.ds(0, row_words)],
                                  sem_o).wait()

        lane = lax.broadcasted_iota(jnp.int32, (_LANES,), 0)
        lane33 = lane * 33

        def _pipelined(n, load, store):
            pipe = [load(i) for i in range(_PRE)]
            for i in range(n):
                if i + _PRE < n:
                    pipe.append(load(i + _PRE))
                store(i, pipe.pop(0))

        def transpose(rbase, obase):
            def s_load(bc):
                return [rows_v[rbase + bc, pl.ds(h * _LANES, _LANES)]
                        for h in range(2)]

            def s_store(bc, v):
                for h in range(2):
                    pad_v[pl.ds(bc * 33 + h * _LANES, _LANES)] = v[h]

            _pipelined(_BC, s_load, s_store)

            def t_load(i):
                d, g = divmod(i, _BC // _LANES)
                return plsc.load_gather(pad_v,
                                        [lane33 + (33 * _LANES * g + d)])

            def t_store(i, vals):
                d, g = divmod(i, _BC // _LANES)
                out_v[pl.ds(obase + d * _BC + _LANES * g, _LANES)] = vals

            _pipelined(32 * (_BC // _LANES), t_load, t_store)

        start_gather(0)

        def group(p, _):
            @pl.when(p + 1 < n_groups)
            def _():
                start_gather(p + 1)

            wait_gather()
            rhalf = lax.rem(p, 2) * g_rows

            def step(q, _):
                s = p * _SG + q
                ob = lax.rem(s, 2)

                @pl.when(s >= 2)
                def _():
                    wait_out()

                transpose(rhalf + q * _BC, ob * row_words)
                start_out(ob, s)
                return ()

            lax.fori_loop(0, _SG, step, ())
            return ()

        lax.fori_loop(0, n_groups, group, ())
        wait_out()
        wait_out()

    return gather_kernel


def kernel(input_ids, table):
    batch, seq = input_ids.shape
    V, D = table.shape
    ids_w = (input_ids.T.reshape(seq, batch // _BC, _BC)
             .transpose(1, 0, 2).reshape(batch // _BC, seq * _BC)
             .astype(jnp.int32))
    out2 = _gather_call(seq, V)(table, ids_w)
    out5 = out2.reshape(seq, _TD, batch // _BC, 8, _BC)
    return out5.transpose(2, 4, 0, 1, 3).reshape(batch, seq, D)


# speedup vs baseline: 1.0023x; 1.0023x over previous
import functools

import jax
import jax.numpy as jnp
from jax import lax
from jax.experimental import pallas as pl
from jax.experimental.pallas import tpu as pltpu
from jax.experimental.pallas import tpu_sc as plsc

_NUM_CORES = 2
_NUM_SUBCORES = 16
_NUM_WORKERS = _NUM_CORES * _NUM_SUBCORES
_LANES = 16
_BC = 128
_TD = 4
_SG = 8
_PRE = 4


def _gather_call(S, V):
    row_words = _TD * 8 * _BC
    g_rows = _SG * _BC
    n_groups = S // _SG
    mesh = plsc.VectorSubcoreMesh(core_axis_name="c", subcore_axis_name="s")

    @functools.partial(
        pl.kernel,
        mesh=mesh,
        out_type=jax.ShapeDtypeStruct((S, _NUM_WORKERS * row_words),
                                      jnp.float32),
        scratch_types=(
            [pltpu.VMEM((S // _SG, _SG * _BC), jnp.int32)]
            + [pltpu.VMEM((2 * g_rows, 32), jnp.float32)]
            + [pltpu.VMEM((2 * row_words,), jnp.float32)]
            + [pltpu.VMEM((_BC * 33,), jnp.float32)]
            + [pltpu.SemaphoreType.DMA for _ in range(2)]
        ),
        compiler_params=pltpu.CompilerParams(use_tc_tiling_on_sc=False,
                                             needs_layout_passes=False),
    )
    def gather_kernel(table_hbm, ids_hbm, out_hbm, idx_all, rows_v, out_v,
                      pad_v, sem_g, sem_o):
        w = lax.axis_index("s") * _NUM_CORES + lax.axis_index("c")

        pltpu.sync_copy(ids_hbm.at[:, w, :], idx_all)

        def start_gather(p):
            half = lax.rem(p, 2) * g_rows
            pltpu.async_copy(
                table_hbm.at[idx_all.at[p]],
                rows_v.at[pl.ds(half, g_rows), :], sem_g)

        def wait_gather():
            pltpu.make_async_copy(
                table_hbm.at[idx_all.at[0]],
                rows_v.at[pl.ds(0, g_rows), :], sem_g).wait()

        def start_out(ob, s):
            for td in range(_TD):
                pltpu.async_copy(
                    out_v.at[pl.ds(ob * row_words + td * 1024, 1024)],
                    out_hbm.at[s, pl.ds(td * _NUM_WORKERS * 1024 + w * 1024,
                                        1024)],
                    sem_o)

        def wait_out():
            pltpu.make_async_copy(out_v.at[pl.ds(0, row_words)],
                                  out_hbm.at[0, pl.ds(0, row_words)],
                                  sem_o).wait()

        lane = lax.broadcasted_iota(jnp.int32, (_LANES,), 0)
        lane33 = lane * 33

        def _pipelined(n, load, store):
            pipe = [load(i) for i in range(_PRE)]
            for i in range(n):
                if i + _PRE < n:
                    pipe.append(load(i + _PRE))
                store(i, pipe.pop(0))

        def transpose(rbase, obase):
            def s_load(bc):
                return [rows_v[rbase + bc, pl.ds(h * _LANES, _LANES)]
                        for h in range(2)]

            def s_store(bc, v):
                for h in range(2):
                    pad_v[pl.ds(bc * 33 + h * _LANES, _LANES)] = v[h]

            _pipelined(_BC, s_load, s_store)

            def t_load(i):
                d, g = divmod(i, _BC // _LANES)
                return plsc.load_gather(pad_v,
                                        [lane33 + (33 * _LANES * g + d)])

            def t_store(i, vals):
                d, g = divmod(i, _BC // _LANES)
                out_v[pl.ds(obase + d * _BC + _LANES * g, _LANES)] = vals

            _pipelined(32 * (_BC // _LANES), t_load, t_store)

        start_gather(0)

        def group(p, _):
            @pl.when(p + 1 < n_groups)
            def _():
                start_gather(p + 1)

            wait_gather()
            rhalf = lax.rem(p, 2) * g_rows

            def step(q, _):
                s = p * _SG + q
                ob = lax.rem(s, 2)

                @pl.when(s >= 2)
                def _():
                    wait_out()

                transpose(rhalf + q * _BC, ob * row_words)
                start_out(ob, s)
                return ()

            lax.fori_loop(0, _SG, step, ())
            return ()

        lax.fori_loop(0, n_groups, group, ())
        wait_out()
        wait_out()

    return gather_kernel


def kernel(input_ids, table):
    batch, seq = input_ids.shape
    V, D = table.shape
    ids4 = (input_ids.astype(jnp.int32)
            .reshape(batch // _BC, _BC, seq // 8, 8)
            .transpose(2, 0, 3, 1).reshape(seq // 8, batch // _BC, 8 * _BC))
    out2 = _gather_call(seq, V)(table, ids4)
    out5 = out2.reshape(seq, _TD, batch // _BC, 8, _BC)
    return out5.transpose(2, 4, 0, 1, 3).reshape(batch, seq, D)
